# baseline (speedup 1.0000x reference)
import jax
import jax.numpy as jnp
from jax.experimental import pallas as pl


def _vq_body(z_ref, cb_ref, cth_ref, zq_ref, idx_ref):
    cb = cb_ref[...]
    k_codes = cb.shape[0]
    hw = z_ref.shape[2]
    half = hw // 2
    cnorm = jnp.sum(cb * cb, axis=1, keepdims=True)
    dn = (((1,), (0,)), ((), ()))

    for h in range(2):
        sl = pl.ds(h * half, half)
        zb = z_ref[0, :, sl]
        s = jax.lax.dot_general(cb, zb, dn, preferred_element_type=jnp.float32)
        znorm = jnp.sum(zb * zb, axis=0, keepdims=True)
        d = (znorm - 2.0 * s) + cnorm

        minv = jnp.min(d, axis=0, keepdims=True)
        ii = jax.lax.broadcasted_iota(jnp.int32, d.shape, 0)
        idx = jnp.min(jnp.where(d == minv, ii, k_codes), axis=0, keepdims=True)

        oh = (ii == idx).astype(jnp.bfloat16)
        zq = jax.lax.dot_general(cth_ref[...], oh, dn,
                                 preferred_element_type=jnp.float32)
        zq_ref[0, :, sl] = zq
        idx_ref[0, :, sl] = idx


def kernel(z, codebook):
    b, c, h, w = z.shape
    hw = h * w
    k = codebook.shape[0]
    z3 = z.reshape(b, c, hw)
    ct_hi = codebook.T.astype(jnp.bfloat16)

    zq3, idx3 = pl.pallas_call(
        _vq_body,
        grid=(b,),
        in_specs=[
            pl.BlockSpec((1, c, hw), lambda i: (i, 0, 0)),
            pl.BlockSpec((k, c), lambda i: (0, 0)),
            pl.BlockSpec((c, k), lambda i: (0, 0)),
        ],
        out_specs=[
            pl.BlockSpec((1, c, hw), lambda i: (i, 0, 0)),
            pl.BlockSpec((1, 1, hw), lambda i: (i, 0, 0)),
        ],
        out_shape=[
            jax.ShapeDtypeStruct((b, c, hw), jnp.float32),
            jax.ShapeDtypeStruct((b, 1, hw), jnp.int32),
        ],
    )(z3, codebook, ct_hi)
    return zq3.reshape(z.shape), idx3.reshape(b, hw)


# speedup vs baseline: 1.0685x; 1.0685x over previous
import jax
import jax.numpy as jnp
from jax.experimental import pallas as pl
from jax.experimental.pallas import tpu as pltpu


def _vq_body(z_ref, cb_ref, cth_ref, zq_ref, idx_ref, prev_ref):
    i = pl.program_id(0)
    nsteps = pl.num_programs(0)
    dn = (((1,), (0,)), ((), ()))
    k_codes = cb_ref.shape[0]

    @pl.when(i > 0)
    def _gather_prev():
        idxp = prev_ref[...]
        iig = jax.lax.broadcasted_iota(jnp.int32, (k_codes, idxp.shape[1]), 0)
        oh = (iig == idxp).astype(jnp.bfloat16)
        zq = jax.lax.dot_general(cth_ref[...], oh, dn,
                                 preferred_element_type=jnp.float32)
        zq_ref[0] = zq
        idx_ref[0] = idxp

    @pl.when(i < nsteps - 1)
    def _score_cur():
        zb = z_ref[0]
        cb = cb_ref[...]
        s = jax.lax.dot_general(cb, zb, dn, preferred_element_type=jnp.float32)
        znorm = jnp.sum(zb * zb, axis=0, keepdims=True)
        cnorm = jnp.sum(cb * cb, axis=1, keepdims=True)
        d = (znorm - 2.0 * s) + cnorm

        minv = jnp.min(d, axis=0, keepdims=True)
        ii = jax.lax.broadcasted_iota(jnp.int32, d.shape, 0)
        idx = jnp.min(jnp.where(d == minv, ii, k_codes), axis=0, keepdims=True)
        prev_ref[...] = idx


def kernel(z, codebook):
    b, c, h, w = z.shape
    hw = h * w
    k = codebook.shape[0]
    z3 = z.reshape(b, c, hw)
    ct_hi = codebook.T.astype(jnp.bfloat16)

    zq3, idx3 = pl.pallas_call(
        _vq_body,
        grid=(b + 1,),
        in_specs=[
            pl.BlockSpec((1, c, hw), lambda i: (jnp.minimum(i, b - 1), 0, 0)),
            pl.BlockSpec((k, c), lambda i: (0, 0)),
            pl.BlockSpec((c, k), lambda i: (0, 0)),
        ],
        out_specs=[
            pl.BlockSpec((1, c, hw), lambda i: (jnp.maximum(i - 1, 0), 0, 0)),
            pl.BlockSpec((1, 1, hw), lambda i: (jnp.maximum(i - 1, 0), 0, 0)),
        ],
        out_shape=[
            jax.ShapeDtypeStruct((b, c, hw), jnp.float32),
            jax.ShapeDtypeStruct((b, 1, hw), jnp.int32),
        ],
        scratch_shapes=[pltpu.VMEM((1, hw), jnp.int32)],
    )(z3, codebook, ct_hi)
    return zq3.reshape(z.shape), idx3.reshape(b, hw)
